# Initial kernel scaffold; baseline (speedup 1.0000x reference)
import jax
import jax.numpy as jnp
from jax.experimental import pallas as pl


def kernel(x, edge_index, W_enc, b_enc, W_msg, b_msg, W_out1, b_out1, W_out2, b_out2):
    raise NotImplementedError("write your pallas kernel here")


# speedup vs baseline: 7.1726x; 7.1726x over previous
import functools

import jax
import jax.numpy as jnp
from jax import lax
from jax.experimental import pallas as pl
from jax.experimental.pallas import tpu as pltpu
from jax.experimental.pallas import tpu_sc as plsc

NC = 2
NS = 16
NW = NC * NS
CH = 80


def _encode_body(x_ref, w_ref, b_ref, o_ref):
    acc = jnp.dot(x_ref[...], w_ref[...], preferred_element_type=jnp.float32)
    o_ref[...] = jnp.maximum(acc + b_ref[...], 0.0)


def _encode(x, w, b2d, block_rows):
    n, d = x.shape
    h = w.shape[1]
    grid = n // block_rows
    return pl.pallas_call(
        _encode_body,
        grid=(grid,),
        in_specs=[
            pl.BlockSpec((block_rows, d), lambda i: (i, 0)),
            pl.BlockSpec((d, h), lambda i: (0, 0)),
            pl.BlockSpec((1, h), lambda i: (0, 0)),
        ],
        out_specs=pl.BlockSpec((block_rows, h), lambda i: (i, 0)),
        out_shape=jax.ShapeDtypeStruct((n, h), jnp.float32),
    )(x, w, b2d)


def _make_sc_segment_sum(n, hdim, nchunk):
    base = (n // NS) // 8 * 8
    rem = n - NS * base
    mesh = plsc.VectorSubcoreMesh(core_axis_name="c", subcore_axis_name="s")

    @functools.partial(
        pl.kernel,
        mesh=mesh,
        out_type=jax.ShapeDtypeStruct((NC, n, hdim), jnp.float32),
        scratch_types=[
            pltpu.VMEM((nchunk, CH), jnp.int32),
            pltpu.VMEM((nchunk, CH), jnp.int32),
            pltpu.VMEM((CH, hdim), jnp.float32),
            pltpu.VMEM_SHARED((n, hdim), jnp.float32),
            pltpu.SemaphoreType.DMA,
        ],
    )
    def sc_seg_sum(h_hbm, src_hbm, dst_hbm, zeros_hbm, out_hbm,
                   src_v, dst_v, rows_v, agg_sh, sem):
        c = lax.axis_index("c")
        s = lax.axis_index("s")
        wid = s * NC + c
        row0 = s * base
        pltpu.sync_copy(zeros_hbm.at[pl.ds(row0, base)],
                        agg_sh.at[pl.ds(row0, base)])

        @pl.when(s == NS - 1)
        def _zero_tail():
            pltpu.sync_copy(zeros_hbm.at[pl.ds(NS * base, rem)],
                            agg_sh.at[pl.ds(NS * base, rem)])
        pltpu.sync_copy(src_hbm.at[wid], src_v)
        pltpu.sync_copy(dst_hbm.at[wid], dst_v)
        plsc.subcore_barrier()

        def body(j, carry):
            pltpu.async_copy(h_hbm.at[src_v.at[j]], rows_v, sem).wait()
            pltpu.sync_copy(rows_v, agg_sh.at[dst_v.at[j]], add=True)
            return carry

        lax.fori_loop(0, nchunk, body, 0)
        plsc.subcore_barrier()
        pltpu.sync_copy(agg_sh.at[pl.ds(row0, base)],
                        out_hbm.at[c, pl.ds(row0, base)])

        @pl.when(s == NS - 1)
        def _out_tail():
            pltpu.sync_copy(agg_sh.at[pl.ds(NS * base, rem)],
                            out_hbm.at[c, pl.ds(NS * base, rem)])

    return sc_seg_sum


def _final_body(p_ref, h_ref, wm_ref, bm_ref, w1_ref, b1_ref, w2_ref, b2_ref,
                o_ref):
    agg = p_ref[0] + p_ref[1]
    t = jnp.dot(agg, wm_ref[...], preferred_element_type=jnp.float32)
    t = jnp.maximum(t + bm_ref[...], 0.0) + h_ref[...]
    hid = jnp.dot(t, w1_ref[...], preferred_element_type=jnp.float32)
    hid = jnp.maximum(hid + b1_ref[...], 0.0)
    o_ref[...] = jnp.dot(hid, w2_ref[...],
                         preferred_element_type=jnp.float32) + b2_ref[...]


def _final(parts, h, wm, bm2d, w1, b12d, w2, b22d, block_rows):
    n, hdim = h.shape
    grid = n // block_rows
    return pl.pallas_call(
        _final_body,
        grid=(grid,),
        in_specs=[
            pl.BlockSpec((NC, block_rows, hdim), lambda i: (0, i, 0)),
            pl.BlockSpec((block_rows, hdim), lambda i: (i, 0)),
            pl.BlockSpec((hdim, hdim), lambda i: (0, 0)),
            pl.BlockSpec((1, hdim), lambda i: (0, 0)),
            pl.BlockSpec((hdim, hdim), lambda i: (0, 0)),
            pl.BlockSpec((1, hdim), lambda i: (0, 0)),
            pl.BlockSpec((hdim, 1), lambda i: (0, 0)),
            pl.BlockSpec((1, 1), lambda i: (0, 0)),
        ],
        out_specs=pl.BlockSpec((block_rows, 1), lambda i: (i, 0)),
        out_shape=jax.ShapeDtypeStruct((n, 1), jnp.float32),
    )(parts, h, wm, bm2d, w1, b12d, w2, b22d)


def kernel(x, edge_index, W_enc, b_enc, W_msg, b_msg, W_out1, b_out1,
           W_out2, b_out2):
    n, d = x.shape
    hdim = W_enc.shape[1]
    e = edge_index.shape[1]
    assert e % (NW * CH) == 0
    nchunk = e // (NW * CH)

    h = _encode(x, W_enc, b_enc.reshape(1, hdim), block_rows=1000)

    src = edge_index[0].reshape(NW, nchunk, CH)
    dst = edge_index[1].reshape(NW, nchunk, CH)
    zeros = jnp.zeros((n, hdim), jnp.float32)
    parts = _make_sc_segment_sum(n, hdim, nchunk)(h, src, dst, zeros)

    return _final(parts, h, W_msg, b_msg.reshape(1, hdim), W_out1,
                  b_out1.reshape(1, hdim), W_out2, b_out2.reshape(1, 1),
                  block_rows=1000)


# speedup vs baseline: 10.4749x; 1.4604x over previous
import functools

import jax
import jax.numpy as jnp
from jax import lax
from jax.experimental import pallas as pl
from jax.experimental.pallas import tpu as pltpu
from jax.experimental.pallas import tpu_sc as plsc

NC = 2
NS = 16
NW = NC * NS
CH = 80


def _encode_body(x_ref, w_ref, b_ref, o_ref):
    acc = jnp.dot(x_ref[...], w_ref[...], preferred_element_type=jnp.float32)
    o_ref[...] = jnp.maximum(acc + b_ref[...], 0.0)


def _encode(x, w, b2d, block_rows):
    n, d = x.shape
    h = w.shape[1]
    grid = n // block_rows
    return pl.pallas_call(
        _encode_body,
        grid=(grid,),
        in_specs=[
            pl.BlockSpec((block_rows, d), lambda i: (i, 0)),
            pl.BlockSpec((d, h), lambda i: (0, 0)),
            pl.BlockSpec((1, h), lambda i: (0, 0)),
        ],
        out_specs=pl.BlockSpec((block_rows, h), lambda i: (i, 0)),
        out_shape=jax.ShapeDtypeStruct((n, h), jnp.float32),
    )(x, w, b2d)


def _make_sc_segment_sum(n, hdim, nchunk):
    base = (n // NS) // 8 * 8
    rem = n - NS * base
    p0 = min(nchunk, (nchunk // 2 + 7) // 8 * 8)
    passes = ((0, p0), (p0, nchunk - p0)) if nchunk > p0 else ((0, p0),)
    idx_rows = max(cnt for _, cnt in passes)
    mesh = plsc.VectorSubcoreMesh(core_axis_name="c", subcore_axis_name="s")

    @functools.partial(
        pl.kernel,
        mesh=mesh,
        out_type=jax.ShapeDtypeStruct((NC, n, hdim), jnp.float32),
        scratch_types=[
            pltpu.VMEM((idx_rows, CH), jnp.int32),
            pltpu.VMEM((idx_rows, CH), jnp.int32),
            pltpu.VMEM((CH, hdim), jnp.float32),
            pltpu.VMEM((CH, hdim), jnp.float32),
            pltpu.VMEM_SHARED((n, hdim), jnp.float32),
            pltpu.SemaphoreType.DMA,
            pltpu.SemaphoreType.DMA,
        ],
    )
    def sc_seg_sum(h_hbm, src_hbm, dst_hbm, zeros_hbm, out_hbm,
                   src_v, dst_v, rows0_v, rows1_v, agg_sh, sem0, sem1):
        c = lax.axis_index("c")
        s = lax.axis_index("s")
        wid = s * NC + c
        row0 = s * base
        pltpu.sync_copy(zeros_hbm.at[pl.ds(row0, base)],
                        agg_sh.at[pl.ds(row0, base)])

        @pl.when(s == NS - 1)
        def _zero_tail():
            pltpu.sync_copy(zeros_hbm.at[pl.ds(NS * base, rem)],
                            agg_sh.at[pl.ds(NS * base, rem)])
        plsc.subcore_barrier()

        bufs = (rows0_v, rows1_v)
        sems = (sem0, sem1)
        nbuf = 2

        for start, cnt in passes:
            pltpu.sync_copy(src_hbm.at[wid, pl.ds(start, cnt)],
                            src_v.at[pl.ds(0, cnt)])
            pltpu.sync_copy(dst_hbm.at[wid, pl.ds(start, cnt)],
                            dst_v.at[pl.ds(0, cnt)])
            for b in range(min(nbuf, cnt)):
                pltpu.async_copy(h_hbm.at[src_v.at[b]], bufs[b], sems[b])

            def body(t, carry):
                j0 = t * nbuf
                for b in range(nbuf):
                    j = j0 + b
                    pltpu.make_async_copy(h_hbm.at[src_v.at[j]], bufs[b],
                                          sems[b]).wait()
                    pltpu.sync_copy(bufs[b], agg_sh.at[dst_v.at[j]], add=True)

                    @pl.when(j + nbuf < cnt)
                    def _next():
                        pltpu.async_copy(h_hbm.at[src_v.at[j + nbuf]],
                                         bufs[b], sems[b])
                return carry

            lax.fori_loop(0, cnt // nbuf, body, 0)
            for j in range(cnt // nbuf * nbuf, cnt):
                b = j % nbuf
                pltpu.make_async_copy(h_hbm.at[src_v.at[j]], bufs[b],
                                      sems[b]).wait()
                pltpu.sync_copy(bufs[b], agg_sh.at[dst_v.at[j]], add=True)
        plsc.subcore_barrier()
        pltpu.sync_copy(agg_sh.at[pl.ds(row0, base)],
                        out_hbm.at[c, pl.ds(row0, base)])

        @pl.when(s == NS - 1)
        def _out_tail():
            pltpu.sync_copy(agg_sh.at[pl.ds(NS * base, rem)],
                            out_hbm.at[c, pl.ds(NS * base, rem)])

    return sc_seg_sum


def _final_body(p_ref, h_ref, wm_ref, bm_ref, w1_ref, b1_ref, w2_ref, b2_ref,
                o_ref):
    agg = p_ref[0] + p_ref[1]
    t = jnp.dot(agg, wm_ref[...], preferred_element_type=jnp.float32)
    t = jnp.maximum(t + bm_ref[...], 0.0) + h_ref[...]
    hid = jnp.dot(t, w1_ref[...], preferred_element_type=jnp.float32)
    hid = jnp.maximum(hid + b1_ref[...], 0.0)
    o_ref[...] = jnp.dot(hid, w2_ref[...],
                         preferred_element_type=jnp.float32) + b2_ref[...]


def _final(parts, h, wm, bm2d, w1, b12d, w2, b22d, block_rows):
    n, hdim = h.shape
    grid = n // block_rows
    return pl.pallas_call(
        _final_body,
        grid=(grid,),
        in_specs=[
            pl.BlockSpec((NC, block_rows, hdim), lambda i: (0, i, 0)),
            pl.BlockSpec((block_rows, hdim), lambda i: (i, 0)),
            pl.BlockSpec((hdim, hdim), lambda i: (0, 0)),
            pl.BlockSpec((1, hdim), lambda i: (0, 0)),
            pl.BlockSpec((hdim, hdim), lambda i: (0, 0)),
            pl.BlockSpec((1, hdim), lambda i: (0, 0)),
            pl.BlockSpec((hdim, 1), lambda i: (0, 0)),
            pl.BlockSpec((1, 1), lambda i: (0, 0)),
        ],
        out_specs=pl.BlockSpec((block_rows, 1), lambda i: (i, 0)),
        out_shape=jax.ShapeDtypeStruct((n, 1), jnp.float32),
    )(parts, h, wm, bm2d, w1, b12d, w2, b22d)


def kernel(x, edge_index, W_enc, b_enc, W_msg, b_msg, W_out1, b_out1,
           W_out2, b_out2):
    n, d = x.shape
    hdim = W_enc.shape[1]
    e = edge_index.shape[1]
    assert e % (NW * CH) == 0
    nchunk = e // (NW * CH)

    h = _encode(x, W_enc, b_enc.reshape(1, hdim), block_rows=1000)

    src = edge_index[0].reshape(NW, nchunk, CH)
    dst = edge_index[1].reshape(NW, nchunk, CH)
    zeros = jnp.zeros((n, hdim), jnp.float32)
    parts = _make_sc_segment_sum(n, hdim, nchunk)(h, src, dst, zeros)

    return _final(parts, h, W_msg, b_msg.reshape(1, hdim), W_out1,
                  b_out1.reshape(1, hdim), W_out2, b_out2.reshape(1, 1),
                  block_rows=1000)


# speedup vs baseline: 10.4770x; 1.0002x over previous
import functools

import jax
import jax.numpy as jnp
from jax import lax
from jax.experimental import pallas as pl
from jax.experimental.pallas import tpu as pltpu
from jax.experimental.pallas import tpu_sc as plsc

NC = 2
NS = 16
NW = NC * NS
CH = 80


def _encode_body(x_ref, w_ref, b_ref, o_ref):
    acc = jnp.dot(x_ref[...], w_ref[...], preferred_element_type=jnp.float32)
    o_ref[...] = jnp.maximum(acc + b_ref[...], 0.0)


def _encode(x, w, b2d, block_rows):
    n, d = x.shape
    h = w.shape[1]
    grid = n // block_rows
    return pl.pallas_call(
        _encode_body,
        grid=(grid,),
        in_specs=[
            pl.BlockSpec((block_rows, d), lambda i: (i, 0)),
            pl.BlockSpec((d, h), lambda i: (0, 0)),
            pl.BlockSpec((1, h), lambda i: (0, 0)),
        ],
        out_specs=pl.BlockSpec((block_rows, h), lambda i: (i, 0)),
        out_shape=jax.ShapeDtypeStruct((n, h), jnp.float32),
    )(x, w, b2d)


def _make_sc_segment_sum(n, hdim, nchunk):
    base = (n // NS) // 8 * 8
    rem = n - NS * base
    p0 = min(nchunk, (nchunk // 2 + 7) // 8 * 8)
    passes = ((0, p0), (p0, nchunk - p0)) if nchunk > p0 else ((0, p0),)
    idx_rows = max(cnt for _, cnt in passes)
    mesh = plsc.VectorSubcoreMesh(core_axis_name="c", subcore_axis_name="s")

    @functools.partial(
        pl.kernel,
        mesh=mesh,
        out_type=jax.ShapeDtypeStruct((NC, n, hdim), jnp.float32),
        scratch_types=[
            pltpu.VMEM((idx_rows, CH), jnp.int32),
            pltpu.VMEM((idx_rows, CH), jnp.int32),
            pltpu.VMEM((CH, hdim), jnp.float32),
            pltpu.VMEM((CH, hdim), jnp.float32),
            pltpu.VMEM_SHARED((n, hdim), jnp.float32),
            pltpu.SemaphoreType.DMA,
            pltpu.SemaphoreType.DMA,
            pltpu.SemaphoreType.DMA,
            pltpu.SemaphoreType.DMA,
        ],
    )
    def sc_seg_sum(h_hbm, src_hbm, dst_hbm, zeros_hbm, out_hbm,
                   src_v, dst_v, rows0_v, rows1_v, agg_sh, sem0, sem1,
                   ssem0, ssem1):
        c = lax.axis_index("c")
        s = lax.axis_index("s")
        wid = s * NC + c
        row0 = s * base
        pltpu.sync_copy(zeros_hbm.at[pl.ds(row0, base)],
                        agg_sh.at[pl.ds(row0, base)])

        @pl.when(s == NS - 1)
        def _zero_tail():
            pltpu.sync_copy(zeros_hbm.at[pl.ds(NS * base, rem)],
                            agg_sh.at[pl.ds(NS * base, rem)])
        plsc.subcore_barrier()

        bufs = (rows0_v, rows1_v)
        gsems = (sem0, sem1)
        ssems = (ssem0, ssem1)
        nbuf = 2

        for start, cnt in passes:
            pltpu.sync_copy(src_hbm.at[wid, pl.ds(start, cnt)],
                            src_v.at[pl.ds(0, cnt)])
            pltpu.sync_copy(dst_hbm.at[wid, pl.ds(start, cnt)],
                            dst_v.at[pl.ds(0, cnt)])
            for b in range(min(nbuf, cnt)):
                pltpu.async_copy(h_hbm.at[src_v.at[b]], bufs[b], gsems[b])

            def body(t, carry):
                j0 = t * nbuf
                for b in range(nbuf):
                    j = j0 + b
                    pltpu.make_async_copy(h_hbm.at[src_v.at[j]], bufs[b],
                                          gsems[b]).wait()
                    pltpu.async_copy(bufs[b], agg_sh.at[dst_v.at[j]],
                                     ssems[b], add=True)

                    @pl.when(j + nbuf < cnt)
                    def _next():
                        pltpu.make_async_copy(bufs[b],
                                              agg_sh.at[dst_v.at[j]],
                                              ssems[b]).wait()
                        pltpu.async_copy(h_hbm.at[src_v.at[j + nbuf]],
                                         bufs[b], gsems[b])
                return carry

            lax.fori_loop(0, cnt // nbuf, body, 0)
            for j in range(cnt // nbuf * nbuf, cnt):
                b = j % nbuf
                pltpu.make_async_copy(h_hbm.at[src_v.at[j]], bufs[b],
                                      gsems[b]).wait()
                pltpu.async_copy(bufs[b], agg_sh.at[dst_v.at[j]],
                                 ssems[b], add=True)
            for j in range(max(0, cnt - nbuf), cnt):
                b = j % nbuf
                pltpu.make_async_copy(bufs[b], agg_sh.at[dst_v.at[j]],
                                      ssems[b]).wait()
        plsc.subcore_barrier()
        pltpu.sync_copy(agg_sh.at[pl.ds(row0, base)],
                        out_hbm.at[c, pl.ds(row0, base)])

        @pl.when(s == NS - 1)
        def _out_tail():
            pltpu.sync_copy(agg_sh.at[pl.ds(NS * base, rem)],
                            out_hbm.at[c, pl.ds(NS * base, rem)])

    return sc_seg_sum


def _final_body(p_ref, h_ref, wm_ref, bm_ref, w1_ref, b1_ref, w2_ref, b2_ref,
                o_ref):
    agg = p_ref[0] + p_ref[1]
    t = jnp.dot(agg, wm_ref[...], preferred_element_type=jnp.float32)
    t = jnp.maximum(t + bm_ref[...], 0.0) + h_ref[...]
    hid = jnp.dot(t, w1_ref[...], preferred_element_type=jnp.float32)
    hid = jnp.maximum(hid + b1_ref[...], 0.0)
    o_ref[...] = jnp.dot(hid, w2_ref[...],
                         preferred_element_type=jnp.float32) + b2_ref[...]


def _final(parts, h, wm, bm2d, w1, b12d, w2, b22d, block_rows):
    n, hdim = h.shape
    grid = n // block_rows
    return pl.pallas_call(
        _final_body,
        grid=(grid,),
        in_specs=[
            pl.BlockSpec((NC, block_rows, hdim), lambda i: (0, i, 0)),
            pl.BlockSpec((block_rows, hdim), lambda i: (i, 0)),
            pl.BlockSpec((hdim, hdim), lambda i: (0, 0)),
            pl.BlockSpec((1, hdim), lambda i: (0, 0)),
            pl.BlockSpec((hdim, hdim), lambda i: (0, 0)),
            pl.BlockSpec((1, hdim), lambda i: (0, 0)),
            pl.BlockSpec((hdim, 1), lambda i: (0, 0)),
            pl.BlockSpec((1, 1), lambda i: (0, 0)),
        ],
        out_specs=pl.BlockSpec((block_rows, 1), lambda i: (i, 0)),
        out_shape=jax.ShapeDtypeStruct((n, 1), jnp.float32),
    )(parts, h, wm, bm2d, w1, b12d, w2, b22d)


def kernel(x, edge_index, W_enc, b_enc, W_msg, b_msg, W_out1, b_out1,
           W_out2, b_out2):
    n, d = x.shape
    hdim = W_enc.shape[1]
    e = edge_index.shape[1]
    assert e % (NW * CH) == 0
    nchunk = e // (NW * CH)

    h = _encode(x, W_enc, b_enc.reshape(1, hdim), block_rows=1000)

    src = edge_index[0].reshape(NW, nchunk, CH)
    dst = edge_index[1].reshape(NW, nchunk, CH)
    zeros = jnp.zeros((n, hdim), jnp.float32)
    parts = _make_sc_segment_sum(n, hdim, nchunk)(h, src, dst, zeros)

    return _final(parts, h, W_msg, b_msg.reshape(1, hdim), W_out1,
                  b_out1.reshape(1, hdim), W_out2, b_out2.reshape(1, 1),
                  block_rows=1000)


# speedup vs baseline: 10.4910x; 1.0013x over previous
import functools

import jax
import jax.numpy as jnp
from jax import lax
from jax.experimental import pallas as pl
from jax.experimental.pallas import tpu as pltpu
from jax.experimental.pallas import tpu_sc as plsc

NC = 2
NS = 16
NW = NC * NS
CH = 80
TRASH = 8


def _encode_body(x_ref, w_ref, b_ref, o_ref):
    acc = jnp.dot(x_ref[...], w_ref[...], preferred_element_type=jnp.float32)
    o_ref[...] = jnp.maximum(acc + b_ref[...], 0.0)


def _encode(x, w, b2d, block_rows):
    n, d = x.shape
    h = w.shape[1]
    grid = n // block_rows
    return pl.pallas_call(
        _encode_body,
        grid=(grid,),
        in_specs=[
            pl.BlockSpec((block_rows, d), lambda i: (i, 0)),
            pl.BlockSpec((d, h), lambda i: (0, 0)),
            pl.BlockSpec((1, h), lambda i: (0, 0)),
        ],
        out_specs=pl.BlockSpec((block_rows, h), lambda i: (i, 0)),
        out_shape=jax.ShapeDtypeStruct((n, h), jnp.float32),
    )(x, w, b2d)


def _make_sc_segment_sum(n, hdim, nchunk):
    base = (n // NS) // 8 * 8
    rem = n - NS * base
    step = 64
    passes = tuple((i, min(step, nchunk - i)) for i in range(0, nchunk, step))
    idx_rows = step
    mesh = plsc.VectorSubcoreMesh(core_axis_name="c", subcore_axis_name="s")

    @functools.partial(
        pl.kernel,
        mesh=mesh,
        out_type=jax.ShapeDtypeStruct((NC, n, hdim), jnp.float32),
        scratch_types=[
            pltpu.VMEM((idx_rows, CH), jnp.int32),
            pltpu.VMEM((idx_rows, CH), jnp.int32),
            pltpu.VMEM((CH, hdim), jnp.float32),
            pltpu.VMEM((CH, hdim), jnp.float32),
            pltpu.VMEM_SHARED((n + TRASH, hdim), jnp.float32),
            pltpu.SemaphoreType.DMA,
            pltpu.SemaphoreType.DMA,
            pltpu.SemaphoreType.DMA,
            pltpu.SemaphoreType.DMA,
        ],
    )
    def sc_seg_sum(h_hbm, src_hbm, dst_hbm, zeros_hbm, out_hbm,
                   src_v, dst_v, rows0_v, rows1_v, agg_sh, sem0, sem1,
                   ssem0, ssem1):
        c = lax.axis_index("c")
        s = lax.axis_index("s")
        wid = s * NC + c
        row0 = s * base
        pltpu.sync_copy(zeros_hbm.at[pl.ds(row0, base)],
                        agg_sh.at[pl.ds(row0, base)])

        @pl.when(s == NS - 1)
        def _zero_tail():
            pltpu.sync_copy(zeros_hbm.at[pl.ds(NS * base, rem)],
                            agg_sh.at[pl.ds(NS * base, rem)])
        plsc.subcore_barrier()

        bufs = (rows0_v, rows1_v)
        gsems = (sem0, sem1)
        ssems = (ssem0, ssem1)
        nbuf = 2

        for start, cnt in passes:
            pltpu.sync_copy(src_hbm.at[wid, pl.ds(start, cnt)],
                            src_v.at[pl.ds(0, cnt)])
            pltpu.sync_copy(dst_hbm.at[wid, pl.ds(start, cnt)],
                            dst_v.at[pl.ds(0, cnt)])
            for b in range(min(nbuf, cnt)):
                pltpu.async_copy(h_hbm.at[src_v.at[b]], bufs[b], gsems[b])

            def body(t, carry):
                j0 = t * nbuf
                for b in range(nbuf):
                    j = j0 + b
                    pltpu.make_async_copy(h_hbm.at[src_v.at[j]], bufs[b],
                                          gsems[b]).wait()
                    pltpu.async_copy(bufs[b], agg_sh.at[dst_v.at[j]],
                                     ssems[b], add=True)

                    @pl.when(j + nbuf < cnt)
                    def _next():
                        pltpu.make_async_copy(bufs[b],
                                              agg_sh.at[dst_v.at[j]],
                                              ssems[b]).wait()
                        pltpu.async_copy(h_hbm.at[src_v.at[j + nbuf]],
                                         bufs[b], gsems[b])
                return carry

            lax.fori_loop(0, cnt // nbuf, body, 0)
            for j in range(cnt // nbuf * nbuf, cnt):
                b = j % nbuf
                pltpu.make_async_copy(h_hbm.at[src_v.at[j]], bufs[b],
                                      gsems[b]).wait()
                pltpu.async_copy(bufs[b], agg_sh.at[dst_v.at[j]],
                                 ssems[b], add=True)
            for j in range(max(0, cnt - nbuf), cnt):
                b = j % nbuf
                pltpu.make_async_copy(bufs[b], agg_sh.at[dst_v.at[j]],
                                      ssems[b]).wait()
        plsc.subcore_barrier()
        pltpu.sync_copy(agg_sh.at[pl.ds(row0, base)],
                        out_hbm.at[c, pl.ds(row0, base)])

        @pl.when(s == NS - 1)
        def _out_tail():
            pltpu.sync_copy(agg_sh.at[pl.ds(NS * base, rem)],
                            out_hbm.at[c, pl.ds(NS * base, rem)])

    return sc_seg_sum


def _final_body(p_ref, h_ref, wm_ref, bm_ref, w1_ref, b1_ref, w2_ref, b2_ref,
                o_ref):
    agg = p_ref[0] + p_ref[1]
    t = jnp.dot(agg, wm_ref[...], preferred_element_type=jnp.float32)
    t = jnp.maximum(t + bm_ref[...], 0.0) + h_ref[...]
    hid = jnp.dot(t, w1_ref[...], preferred_element_type=jnp.float32)
    hid = jnp.maximum(hid + b1_ref[...], 0.0)
    o_ref[...] = jnp.dot(hid, w2_ref[...],
                         preferred_element_type=jnp.float32) + b2_ref[...]


def _final(parts, h, wm, bm2d, w1, b12d, w2, b22d, block_rows):
    n, hdim = h.shape
    grid = n // block_rows
    return pl.pallas_call(
        _final_body,
        grid=(grid,),
        in_specs=[
            pl.BlockSpec((NC, block_rows, hdim), lambda i: (0, i, 0)),
            pl.BlockSpec((block_rows, hdim), lambda i: (i, 0)),
            pl.BlockSpec((hdim, hdim), lambda i: (0, 0)),
            pl.BlockSpec((1, hdim), lambda i: (0, 0)),
            pl.BlockSpec((hdim, hdim), lambda i: (0, 0)),
            pl.BlockSpec((1, hdim), lambda i: (0, 0)),
            pl.BlockSpec((hdim, 1), lambda i: (0, 0)),
            pl.BlockSpec((1, 1), lambda i: (0, 0)),
        ],
        out_specs=pl.BlockSpec((block_rows, 1), lambda i: (i, 0)),
        out_shape=jax.ShapeDtypeStruct((n, 1), jnp.float32),
    )(parts, h, wm, bm2d, w1, b12d, w2, b22d)


def kernel(x, edge_index, W_enc, b_enc, W_msg, b_msg, W_out1, b_out1,
           W_out2, b_out2):
    n, d = x.shape
    hdim = W_enc.shape[1]
    e = edge_index.shape[1]
    assert e % (NW * CH) == 0
    nchunk = e // (NW * CH)

    h = _encode(x, W_enc, b_enc.reshape(1, hdim), block_rows=1000)

    src = edge_index[0].reshape(NW, nchunk, CH)
    dst = edge_index[1].reshape(NW, nchunk, CH)
    zeros = jnp.zeros((n, hdim), jnp.float32)
    parts = _make_sc_segment_sum(n, hdim, nchunk)(h, src, dst, zeros)

    return _final(parts, h, W_msg, b_msg.reshape(1, hdim), W_out1,
                  b_out1.reshape(1, hdim), W_out2, b_out2.reshape(1, 1),
                  block_rows=1000)
